# baseline (speedup 1.0000x reference)
import jax
import jax.numpy as jnp
from jax.experimental import pallas as pl
from jax.experimental.pallas import tpu as pltpu
from jax.experimental.pallas import tpu_sc as plsc

_SEP = 102
_N = 32768
_D = 2048
_BLK = 1024
_SUBCORES = 16
_PER_SUB = _N // _SUBCORES
_LANES = 16


def _sc_scan(x):
    xr = x.reshape(_SUBCORES, _PER_SUB)

    @pl.kernel(
        out_type=jax.ShapeDtypeStruct((_SUBCORES, _LANES), jnp.int32),
        mesh=plsc.VectorSubcoreMesh(
            core_axis_name="c", subcore_axis_name="s", num_cores=1
        ),
        scratch_types=[
            pltpu.VMEM((_PER_SUB,), jnp.int32),
            pltpu.VMEM((_LANES,), jnp.int32),
        ],
    )
    def scan_kernel(x_hbm, o_hbm, xv, accv):
        k = jax.lax.axis_index("s")
        pltpu.sync_copy(x_hbm.at[k], xv)
        lane = jax.lax.iota(jnp.int32, _LANES)
        base = k * _PER_SUB

        def body(i, acc):
            v = xv[pl.ds(i * _LANES, _LANES)]
            pos = base + i * _LANES + lane
            return jnp.minimum(acc, jnp.where(v == _SEP, pos, _N))

        accv[...] = jax.lax.fori_loop(
            0, _PER_SUB // _LANES, body, jnp.full((_LANES,), _N, jnp.int32)
        )
        pltpu.sync_copy(accv, o_hbm.at[k])

    return scan_kernel(xr)


def _write_body(part_ref, tab_ref, out_ref, t_ref):
    i = pl.program_id(0)

    @pl.when(i == 0)
    def _():
        t_ref[0] = jnp.min(part_ref[...])

    t = t_ref[0]
    row = i * _BLK + jax.lax.broadcasted_iota(jnp.int32, (_BLK, _D), 0)
    t0 = jnp.broadcast_to(tab_ref[0:1, :], (_BLK, _D))
    t1 = jnp.broadcast_to(tab_ref[1:2, :], (_BLK, _D))
    out_ref[...] = jnp.where(row >= t, t1, t0)


def kernel(x, table):
    partials = _sc_scan(x)
    out = pl.pallas_call(
        _write_body,
        grid=(_N // _BLK,),
        in_specs=[
            pl.BlockSpec(partials.shape, lambda i: (0, 0)),
            pl.BlockSpec(table.shape, lambda i: (0, 0)),
        ],
        out_specs=pl.BlockSpec((_BLK, _D), lambda i: (i, 0)),
        out_shape=jax.ShapeDtypeStruct((_N, _D), table.dtype),
        scratch_shapes=[pltpu.SMEM((1,), jnp.int32)],
    )(partials, table)
    return out.reshape(x.shape + (table.shape[1],))
